# baseline (speedup 1.0000x reference)
import functools

import jax
import jax.numpy as jnp
from jax import lax
from jax.experimental import pallas as pl
from jax.experimental.pallas import tpu as pltpu
from jax.experimental.pallas import tpu_sc as plsc

N_P = 10000
D = 128
E = 320000

NC = 2
NS = 16
NW = NC * NS
CHUNK = 128
N_CHUNKS = -(-E // CHUNK // NW) * NW
E_PAD = N_CHUNKS * CHUNK
ITERS = N_CHUNKS // NW
N_REAL = -(-E // CHUNK)
ROWS_PAD = 10240
ROWS_PER_TILE = ROWS_PAD // NS
CNT_PAD = 10240
CNT_PER_TILE = CNT_PAD // NS


def _mesh():
    return plsc.VectorSubcoreMesh(core_axis_name="c", subcore_axis_name="s")


def _sc_scatter_rows(y, src, dst, zrows):
    @functools.partial(
        pl.kernel,
        out_type=jax.ShapeDtypeStruct((NC, ROWS_PAD, D), jnp.float32),
        mesh=_mesh(),
        scratch_types=[
            pltpu.VMEM((CHUNK,), jnp.int32),
            pltpu.VMEM((CHUNK,), jnp.int32),
            pltpu.VMEM((CHUNK,), jnp.int32),
            pltpu.VMEM((CHUNK,), jnp.int32),
            pltpu.VMEM((CHUNK, D), jnp.float32),
            pltpu.VMEM((CHUNK, D), jnp.float32),
            pltpu.VMEM_SHARED((ROWS_PAD, D), jnp.float32),
            pltpu.SemaphoreType.DMA,
            pltpu.SemaphoreType.DMA,
            pltpu.SemaphoreType.DMA,
            pltpu.SemaphoreType.DMA,
            pltpu.SemaphoreType.DMA,
            pltpu.SemaphoreType.DMA,
            pltpu.SemaphoreType.DMA,
            pltpu.SemaphoreType.DMA,
        ],
    )
    def run(y_hbm, src_hbm, dst_hbm, z_hbm, out_hbm,
            src_v0, dst_v0, src_v1, dst_v1, rows_v0, rows_v1, acc_sh,
            es0, ed0, es1, ed1, eg0, eg1, ss0, ss1):
        cid = lax.axis_index("c")
        sid = lax.axis_index("s")
        wid = cid * NS + sid
        r0 = sid * ROWS_PER_TILE
        pltpu.sync_copy(z_hbm, acc_sh.at[pl.ds(r0, ROWS_PER_TILE)])
        plsc.subcore_barrier()

        sets = ((src_v0, dst_v0, rows_v0, es0, ed0, eg0, ss0),
                (src_v1, dst_v1, rows_v1, es1, ed1, eg1, ss1))

        def issue_idx(i, b):
            sv, dv, rv, es, ed, eg, ss = sets[b]
            c = i * NW + wid

            @pl.when(c < N_REAL)
            def _():
                base = c * CHUNK
                pltpu.async_copy(src_hbm.at[pl.ds(base, CHUNK)], sv, es)
                pltpu.async_copy(dst_hbm.at[pl.ds(base, CHUNK)], dv, ed)

        def issue_gather(i, b):
            sv, dv, rv, es, ed, eg, ss = sets[b]
            c = i * NW + wid

            @pl.when(c < N_REAL)
            def _():
                base = c * CHUNK
                pltpu.make_async_copy(src_hbm.at[pl.ds(base, CHUNK)], sv, es).wait()
                pltpu.make_async_copy(dst_hbm.at[pl.ds(base, CHUNK)], dv, ed).wait()
                pltpu.async_copy(y_hbm.at[sv], rv, eg)

        def scatter_async(i, b):
            sv, dv, rv, es, ed, eg, ss = sets[b]
            c = i * NW + wid

            @pl.when(c < N_REAL)
            def _():
                pltpu.make_async_copy(y_hbm.at[sv], rv, eg).wait()
                pltpu.async_copy(rv, acc_sh.at[dv], ss, add=True)

        def wait_scatter(i, b):
            sv, dv, rv, es, ed, eg, ss = sets[b]
            c = i * NW + wid

            @pl.when(c < N_REAL)
            def _():
                pltpu.make_async_copy(rv, acc_sh.at[dv], ss).wait()

        issue_idx(0, 0)
        issue_idx(1, 1)
        issue_gather(0, 0)

        @pl.loop(0, ITERS + 1, step=2)
        def _(i):
            issue_gather(i + 1, 1)
            scatter_async(i, 0)
            scatter_async(i + 1, 1)
            wait_scatter(i, 0)
            issue_idx(i + 2, 0)
            issue_gather(i + 2, 0)
            wait_scatter(i + 1, 1)
            issue_idx(i + 3, 1)

        plsc.subcore_barrier()
        pltpu.sync_copy(acc_sh.at[pl.ds(r0, ROWS_PER_TILE)],
                        out_hbm.at[cid, pl.ds(r0, ROWS_PER_TILE)])

    return run(y, src, dst, zrows)


def _sc_counts_agg(xa, dst_c, src_w, dst_w, zrows, zvec, ones):
    out_types = (
        jax.ShapeDtypeStruct((NC, CNT_PAD), jnp.float32),
        jax.ShapeDtypeStruct((NC, CNT_PAD), jnp.float32),
        jax.ShapeDtypeStruct((NC, ROWS_PAD, D), jnp.float32),
    )

    @functools.partial(
        pl.kernel,
        out_type=out_types,
        mesh=_mesh(),
        scratch_types=[
            pltpu.VMEM((CHUNK,), jnp.int32),
            pltpu.VMEM((CHUNK,), jnp.int32),
            pltpu.VMEM((CHUNK,), jnp.int32),
            pltpu.VMEM((CHUNK,), jnp.int32),
            pltpu.VMEM((CHUNK,), jnp.int32),
            pltpu.VMEM((CHUNK,), jnp.int32),
            pltpu.VMEM((CHUNK, D), jnp.float32),
            pltpu.VMEM((CHUNK, D), jnp.float32),
            pltpu.VMEM((CHUNK,), jnp.float32),
            pltpu.VMEM_SHARED((CNT_PAD,), jnp.float32),
            pltpu.VMEM_SHARED((CNT_PAD,), jnp.float32),
            pltpu.VMEM_SHARED((ROWS_PAD, D), jnp.float32),
            pltpu.SemaphoreType.DMA,
            pltpu.SemaphoreType.DMA,
            pltpu.SemaphoreType.DMA,
            pltpu.SemaphoreType.DMA,
            pltpu.SemaphoreType.DMA,
            pltpu.SemaphoreType.DMA,
            pltpu.SemaphoreType.DMA,
            pltpu.SemaphoreType.DMA,
        ],
    )
    def run(xa_hbm, dstc_hbm, srcw_hbm, dstw_hbm, z_hbm, zv_hbm, ones_hbm,
            odeg_hbm, ocnt_hbm, oagg_hbm,
            dca0, swv0, dwv0, dca1, swv1, dwv1, rows_v0, rows_v1, ones_v,
            acc_deg, acc_cnt, acc_agg,
            ea0, eb0, ec0, ea1, eb1, ec1, eg0, eg1):
        cid = lax.axis_index("c")
        sid = lax.axis_index("s")
        wid = cid * NS + sid
        r0 = sid * ROWS_PER_TILE
        c0 = sid * CNT_PER_TILE
        pltpu.sync_copy(ones_hbm, ones_v)
        pltpu.sync_copy(z_hbm, acc_agg.at[pl.ds(r0, ROWS_PER_TILE)])
        pltpu.sync_copy(zv_hbm, acc_deg.at[pl.ds(c0, CNT_PER_TILE)])
        pltpu.sync_copy(zv_hbm, acc_cnt.at[pl.ds(c0, CNT_PER_TILE)])
        plsc.subcore_barrier()

        sets = ((dca0, swv0, dwv0, rows_v0, ea0, eb0, ec0, eg0),
                (dca1, swv1, dwv1, rows_v1, ea1, eb1, ec1, eg1))

        def issue_idx(i, b):
            dca, swv, dwv, rv, ea, eb, ec, eg = sets[b]
            c = i * NW + wid

            @pl.when(c < N_REAL)
            def _():
                base = c * CHUNK
                pltpu.async_copy(dstc_hbm.at[pl.ds(base, CHUNK)], dca, ea)
                pltpu.async_copy(srcw_hbm.at[pl.ds(base, CHUNK)], swv, eb)
                pltpu.async_copy(dstw_hbm.at[pl.ds(base, CHUNK)], dwv, ec)

        def issue_gather(i, b):
            dca, swv, dwv, rv, ea, eb, ec, eg = sets[b]
            c = i * NW + wid

            @pl.when(c < N_REAL)
            def _():
                base = c * CHUNK
                pltpu.make_async_copy(srcw_hbm.at[pl.ds(base, CHUNK)], swv, eb).wait()
                pltpu.async_copy(xa_hbm.at[swv], rv, eg)

        def deg_scatter(i, b):
            dca, swv, dwv, rv, ea, eb, ec, eg = sets[b]
            c = i * NW + wid

            @pl.when(c < N_REAL)
            def _():
                base = c * CHUNK
                pltpu.make_async_copy(dstc_hbm.at[pl.ds(base, CHUNK)], dca, ea).wait()
                pltpu.sync_copy(ones_v, acc_deg.at[dca], add=True)

        def work(i, b):
            dca, swv, dwv, rv, ea, eb, ec, eg = sets[b]
            c = i * NW + wid

            @pl.when(c < N_REAL)
            def _():
                base = c * CHUNK
                pltpu.make_async_copy(dstw_hbm.at[pl.ds(base, CHUNK)], dwv, ec).wait()
                pltpu.make_async_copy(xa_hbm.at[swv], rv, eg).wait()
                pltpu.sync_copy(rv, acc_agg.at[dwv], add=True)
                pltpu.sync_copy(ones_v, acc_cnt.at[dwv], add=True)

        issue_idx(0, 0)
        issue_idx(1, 1)
        issue_gather(0, 0)

        @pl.loop(0, ITERS + 1, step=2)
        def _(i):
            issue_gather(i + 1, 1)
            deg_scatter(i, 0)
            work(i, 0)
            issue_idx(i + 2, 0)
            issue_gather(i + 2, 0)
            deg_scatter(i + 1, 1)
            work(i + 1, 1)
            issue_idx(i + 3, 1)

        plsc.subcore_barrier()
        pltpu.sync_copy(acc_deg.at[pl.ds(c0, CNT_PER_TILE)],
                        odeg_hbm.at[cid, pl.ds(c0, CNT_PER_TILE)])
        pltpu.sync_copy(acc_cnt.at[pl.ds(c0, CNT_PER_TILE)],
                        ocnt_hbm.at[cid, pl.ds(c0, CNT_PER_TILE)])
        pltpu.sync_copy(acc_agg.at[pl.ds(r0, ROWS_PER_TILE)],
                        oagg_hbm.at[cid, pl.ds(r0, ROWS_PER_TILE)])

    return run(xa, dst_c, src_w, dst_w, zrows, zvec, ones)


R_BLK = 2000


def _row(i):
    return (i, 0)


def _zero(i):
    return (0, 0)


def _rspec():
    return pl.BlockSpec((R_BLK, D), _row)


def _cspec():
    return pl.BlockSpec((R_BLK, 1), _row)


def _wspec():
    return pl.BlockSpec((D, D), _zero)


def _bspec():
    return pl.BlockSpec((1, D), _zero)


def _tc_y1_body(x_ref, w_ref, d0_ref, d1_ref, o_ref):
    dis = lax.rsqrt(d0_ref[...] + d1_ref[...] + 1.0)
    o_ref[...] = jnp.dot(x_ref[...], w_ref[...],
                         preferred_element_type=jnp.float32) * dis


def _tc_y1(x, W, d0, d1):
    return pl.pallas_call(
        _tc_y1_body,
        grid=(N_P // R_BLK,),
        in_specs=[_rspec(), _wspec(), _cspec(), _cspec()],
        out_specs=_rspec(),
        out_shape=jax.ShapeDtypeStruct((N_P, D), jnp.float32),
    )(x, W, d0, d1)


def _tc_mid_body(s10_ref, s11_ref, y1_ref, d0_ref, d1_ref, a0_ref, a1_ref,
                 c0_ref, c1_ref, x_ref, wl_ref, wr_ref, wg2_ref, bg1_ref,
                 bs1_ref, y2_ref):
    dis = lax.rsqrt(d0_ref[...] + d1_ref[...] + 1.0)
    gcn1 = (s10_ref[...] + s11_ref[...] + y1_ref[...]) * dis + bg1_ref[...]
    mean = (a0_ref[...] + a1_ref[...]) / jnp.maximum(c0_ref[...] + c1_ref[...], 1.0)
    sage = (jnp.dot(mean, wl_ref[...], preferred_element_type=jnp.float32)
            + jnp.dot(x_ref[...], wr_ref[...], preferred_element_type=jnp.float32)
            + bs1_ref[...])
    h = jnp.maximum(gcn1 + sage, 0.0)
    y2_ref[...] = jnp.dot(h, wg2_ref[...],
                          preferred_element_type=jnp.float32) * dis


def _tc_mid(s10, s11, y1, d0, d1, a0, a1, c0, c1, x, wl, wr, wg2, bg1, bs1):
    return pl.pallas_call(
        _tc_mid_body,
        grid=(N_P // R_BLK,),
        in_specs=[_rspec(), _rspec(), _rspec(), _cspec(), _cspec(),
                  _rspec(), _rspec(), _cspec(), _cspec(), _rspec(),
                  _wspec(), _wspec(), _wspec(), _bspec(), _bspec()],
        out_specs=_rspec(),
        out_shape=jax.ShapeDtypeStruct((N_P, D), jnp.float32),
    )(s10, s11, y1, d0, d1, a0, a1, c0, c1, x, wl, wr, wg2, bg1, bs1)


def _tc_out_body(s20_ref, s21_ref, y2_ref, d0_ref, d1_ref, lw_ref, bg2_ref,
                 lb_ref, o_ref):
    dis = lax.rsqrt(d0_ref[...] + d1_ref[...] + 1.0)
    h2 = jnp.maximum((s20_ref[...] + s21_ref[...] + y2_ref[...]) * dis
                     + bg2_ref[...], 0.0)
    o_ref[...] = jnp.dot(h2, lw_ref[...],
                         preferred_element_type=jnp.float32) + lb_ref[...]


def _tc_out(s20, s21, y2, d0, d1, lw, bg2, lb):
    return pl.pallas_call(
        _tc_out_body,
        grid=(N_P // R_BLK,),
        in_specs=[_rspec(), _rspec(), _rspec(), _cspec(), _cspec(),
                  _wspec(), _bspec(), _bspec()],
        out_specs=_rspec(),
        out_shape=jax.ShapeDtypeStruct((N_P, D), jnp.float32),
    )(s20, s21, y2, d0, d1, lw, bg2, lb)


def kernel(x_paper, x_author, edge_index_cites, edge_index_writes,
           W_gcn1, b_gcn1, Wl1, Wr1, b_s1, W_gcn2, b_gcn2, lin_W, lin_b):
    pad_src = jnp.zeros((E_PAD - E,), jnp.int32)
    pad_dst = N_P + jnp.arange(E_PAD - E, dtype=jnp.int32) % (ROWS_PAD - N_P)
    src_c = jnp.concatenate([edge_index_cites[0], pad_src])
    dst_c = jnp.concatenate([edge_index_cites[1], pad_dst])
    src_w = jnp.concatenate([edge_index_writes[0], pad_src])
    dst_w = jnp.concatenate([edge_index_writes[1], pad_dst])

    zrows = jnp.zeros((ROWS_PER_TILE, D), jnp.float32)
    zvec = jnp.zeros((CNT_PER_TILE,), jnp.float32)
    ones = jnp.ones((CHUNK,), jnp.float32)

    deg_p, cnt_p, agg_p = _sc_counts_agg(x_author, dst_c, src_w, dst_w,
                                         zrows, zvec, ones)
    d0 = deg_p[0, :N_P].reshape(N_P, 1)
    d1 = deg_p[1, :N_P].reshape(N_P, 1)
    c0 = cnt_p[0, :N_P].reshape(N_P, 1)
    c1 = cnt_p[1, :N_P].reshape(N_P, 1)

    y1 = _tc_y1(x_paper, W_gcn1, d0, d1)
    s1_p = _sc_scatter_rows(y1, src_c, dst_c, zrows)
    y2 = _tc_mid(s1_p[0, :N_P], s1_p[1, :N_P], y1, d0, d1,
                 agg_p[0, :N_P], agg_p[1, :N_P], c0, c1,
                 x_paper, Wl1, Wr1, W_gcn2,
                 b_gcn1.reshape(1, D), b_s1.reshape(1, D))
    s2_p = _sc_scatter_rows(y2, src_c, dst_c, zrows)
    return _tc_out(s2_p[0, :N_P], s2_p[1, :N_P], y2, d0, d1, lin_W,
                   b_gcn2.reshape(1, D), lin_b.reshape(1, D))


# speedup vs baseline: 1.0487x; 1.0487x over previous
import functools

import jax
import jax.numpy as jnp
from jax import lax
from jax.experimental import pallas as pl
from jax.experimental.pallas import tpu as pltpu
from jax.experimental.pallas import tpu_sc as plsc

N_P = 10000
D = 128
E = 320000

NC = 2
NS = 16
NW = NC * NS
CHUNK = 128
N_CHUNKS = -(-E // CHUNK // NW) * NW
E_PAD = N_CHUNKS * CHUNK
ITERS = N_CHUNKS // NW
N_REAL = -(-E // CHUNK)
ROWS_PAD = 10240
ROWS_PER_TILE = ROWS_PAD // NS
CNT_PAD = 10240
CNT_PER_TILE = CNT_PAD // NS


def _mesh():
    return plsc.VectorSubcoreMesh(core_axis_name="c", subcore_axis_name="s")


def _sc_scatter_rows(y, src, dst, zrows):
    @functools.partial(
        pl.kernel,
        out_type=jax.ShapeDtypeStruct((NC, ROWS_PAD, D), jnp.float32),
        mesh=_mesh(),
        scratch_types=[
            pltpu.VMEM((CHUNK,), jnp.int32),
            pltpu.VMEM((CHUNK,), jnp.int32),
            pltpu.VMEM((CHUNK,), jnp.int32),
            pltpu.VMEM((CHUNK,), jnp.int32),
            pltpu.VMEM((CHUNK, D), jnp.float32),
            pltpu.VMEM((CHUNK, D), jnp.float32),
            pltpu.VMEM_SHARED((ROWS_PAD, D), jnp.float32),
            pltpu.SemaphoreType.DMA,
            pltpu.SemaphoreType.DMA,
            pltpu.SemaphoreType.DMA,
            pltpu.SemaphoreType.DMA,
            pltpu.SemaphoreType.DMA,
            pltpu.SemaphoreType.DMA,
            pltpu.SemaphoreType.DMA,
            pltpu.SemaphoreType.DMA,
        ],
    )
    def run(y_hbm, src_hbm, dst_hbm, z_hbm, out_hbm,
            src_v0, dst_v0, src_v1, dst_v1, rows_v0, rows_v1, acc_sh,
            es0, ed0, es1, ed1, eg0, eg1, ss0, ss1):
        cid = lax.axis_index("c")
        sid = lax.axis_index("s")
        wid = cid * NS + sid
        r0 = sid * ROWS_PER_TILE
        pltpu.sync_copy(z_hbm, acc_sh.at[pl.ds(r0, ROWS_PER_TILE)])
        plsc.subcore_barrier()

        sets = ((src_v0, dst_v0, rows_v0, es0, ed0, eg0, ss0),
                (src_v1, dst_v1, rows_v1, es1, ed1, eg1, ss1))

        def issue_idx(i, b):
            sv, dv, rv, es, ed, eg, ss = sets[b]
            c = i * NW + wid

            @pl.when(c < N_REAL)
            def _():
                base = c * CHUNK
                pltpu.async_copy(src_hbm.at[pl.ds(base, CHUNK)], sv, es)
                pltpu.async_copy(dst_hbm.at[pl.ds(base, CHUNK)], dv, ed)

        def issue_gather(i, b):
            sv, dv, rv, es, ed, eg, ss = sets[b]
            c = i * NW + wid

            @pl.when(c < N_REAL)
            def _():
                base = c * CHUNK
                pltpu.make_async_copy(src_hbm.at[pl.ds(base, CHUNK)], sv, es).wait()
                pltpu.make_async_copy(dst_hbm.at[pl.ds(base, CHUNK)], dv, ed).wait()
                pltpu.async_copy(y_hbm.at[sv], rv, eg)

        def scatter(i, b):
            sv, dv, rv, es, ed, eg, ss = sets[b]
            c = i * NW + wid

            @pl.when(c < N_REAL)
            def _():
                pltpu.make_async_copy(y_hbm.at[sv], rv, eg).wait()
                pltpu.sync_copy(rv, acc_sh.at[dv], add=True)

        issue_idx(0, 0)
        issue_idx(1, 1)
        issue_gather(0, 0)

        @pl.loop(0, ITERS + 1, step=2)
        def _(i):
            issue_gather(i + 1, 1)
            scatter(i, 0)
            issue_idx(i + 2, 0)
            issue_gather(i + 2, 0)
            scatter(i + 1, 1)
            issue_idx(i + 3, 1)

        plsc.subcore_barrier()
        pltpu.sync_copy(acc_sh.at[pl.ds(r0, ROWS_PER_TILE)],
                        out_hbm.at[cid, pl.ds(r0, ROWS_PER_TILE)])

    return run(y, src, dst, zrows)


def _sc_counts_agg(xa, dst_c, src_w, dst_w, zrows, zvec, ones):
    out_types = (
        jax.ShapeDtypeStruct((NC, CNT_PAD), jnp.float32),
        jax.ShapeDtypeStruct((NC, CNT_PAD), jnp.float32),
        jax.ShapeDtypeStruct((NC, ROWS_PAD, D), jnp.float32),
    )

    @functools.partial(
        pl.kernel,
        out_type=out_types,
        mesh=_mesh(),
        scratch_types=[
            pltpu.VMEM((CHUNK,), jnp.int32),
            pltpu.VMEM((CHUNK,), jnp.int32),
            pltpu.VMEM((CHUNK,), jnp.int32),
            pltpu.VMEM((CHUNK,), jnp.int32),
            pltpu.VMEM((CHUNK,), jnp.int32),
            pltpu.VMEM((CHUNK,), jnp.int32),
            pltpu.VMEM((CHUNK, D), jnp.float32),
            pltpu.VMEM((CHUNK, D), jnp.float32),
            pltpu.VMEM((CHUNK,), jnp.float32),
            pltpu.VMEM_SHARED((CNT_PAD,), jnp.float32),
            pltpu.VMEM_SHARED((CNT_PAD,), jnp.float32),
            pltpu.VMEM_SHARED((ROWS_PAD, D), jnp.float32),
            pltpu.SemaphoreType.DMA,
            pltpu.SemaphoreType.DMA,
            pltpu.SemaphoreType.DMA,
            pltpu.SemaphoreType.DMA,
            pltpu.SemaphoreType.DMA,
            pltpu.SemaphoreType.DMA,
            pltpu.SemaphoreType.DMA,
            pltpu.SemaphoreType.DMA,
        ],
    )
    def run(xa_hbm, dstc_hbm, srcw_hbm, dstw_hbm, z_hbm, zv_hbm, ones_hbm,
            odeg_hbm, ocnt_hbm, oagg_hbm,
            dca0, swv0, dwv0, dca1, swv1, dwv1, rows_v0, rows_v1, ones_v,
            acc_deg, acc_cnt, acc_agg,
            ea0, eb0, ec0, ea1, eb1, ec1, eg0, eg1):
        cid = lax.axis_index("c")
        sid = lax.axis_index("s")
        wid = cid * NS + sid
        r0 = sid * ROWS_PER_TILE
        c0 = sid * CNT_PER_TILE
        pltpu.sync_copy(ones_hbm, ones_v)
        pltpu.sync_copy(z_hbm, acc_agg.at[pl.ds(r0, ROWS_PER_TILE)])
        pltpu.sync_copy(zv_hbm, acc_deg.at[pl.ds(c0, CNT_PER_TILE)])
        pltpu.sync_copy(zv_hbm, acc_cnt.at[pl.ds(c0, CNT_PER_TILE)])
        plsc.subcore_barrier()

        sets = ((dca0, swv0, dwv0, rows_v0, ea0, eb0, ec0, eg0),
                (dca1, swv1, dwv1, rows_v1, ea1, eb1, ec1, eg1))

        def issue_idx(i, b):
            dca, swv, dwv, rv, ea, eb, ec, eg = sets[b]
            c = i * NW + wid

            @pl.when(c < N_REAL)
            def _():
                base = c * CHUNK
                pltpu.async_copy(dstc_hbm.at[pl.ds(base, CHUNK)], dca, ea)
                pltpu.async_copy(srcw_hbm.at[pl.ds(base, CHUNK)], swv, eb)
                pltpu.async_copy(dstw_hbm.at[pl.ds(base, CHUNK)], dwv, ec)

        def issue_gather(i, b):
            dca, swv, dwv, rv, ea, eb, ec, eg = sets[b]
            c = i * NW + wid

            @pl.when(c < N_REAL)
            def _():
                base = c * CHUNK
                pltpu.make_async_copy(srcw_hbm.at[pl.ds(base, CHUNK)], swv, eb).wait()
                pltpu.async_copy(xa_hbm.at[swv], rv, eg)

        def deg_scatter(i, b):
            dca, swv, dwv, rv, ea, eb, ec, eg = sets[b]
            c = i * NW + wid

            @pl.when(c < N_REAL)
            def _():
                base = c * CHUNK
                pltpu.make_async_copy(dstc_hbm.at[pl.ds(base, CHUNK)], dca, ea).wait()
                pltpu.sync_copy(ones_v, acc_deg.at[dca], add=True)

        def work(i, b):
            dca, swv, dwv, rv, ea, eb, ec, eg = sets[b]
            c = i * NW + wid

            @pl.when(c < N_REAL)
            def _():
                base = c * CHUNK
                pltpu.make_async_copy(dstw_hbm.at[pl.ds(base, CHUNK)], dwv, ec).wait()
                pltpu.make_async_copy(xa_hbm.at[swv], rv, eg).wait()
                pltpu.sync_copy(rv, acc_agg.at[dwv], add=True)
                pltpu.sync_copy(ones_v, acc_cnt.at[dwv], add=True)

        issue_idx(0, 0)
        issue_idx(1, 1)
        issue_gather(0, 0)

        @pl.loop(0, ITERS + 1, step=2)
        def _(i):
            issue_gather(i + 1, 1)
            deg_scatter(i, 0)
            work(i, 0)
            issue_idx(i + 2, 0)
            issue_gather(i + 2, 0)
            deg_scatter(i + 1, 1)
            work(i + 1, 1)
            issue_idx(i + 3, 1)

        plsc.subcore_barrier()
        pltpu.sync_copy(acc_deg.at[pl.ds(c0, CNT_PER_TILE)],
                        odeg_hbm.at[cid, pl.ds(c0, CNT_PER_TILE)])
        pltpu.sync_copy(acc_cnt.at[pl.ds(c0, CNT_PER_TILE)],
                        ocnt_hbm.at[cid, pl.ds(c0, CNT_PER_TILE)])
        pltpu.sync_copy(acc_agg.at[pl.ds(r0, ROWS_PER_TILE)],
                        oagg_hbm.at[cid, pl.ds(r0, ROWS_PER_TILE)])

    return run(xa, dst_c, src_w, dst_w, zrows, zvec, ones)


R_BLK = 2000


def _row(i):
    return (i, 0)


def _zero(i):
    return (0, 0)


def _rspec():
    return pl.BlockSpec((R_BLK, D), _row)


def _cspec():
    return pl.BlockSpec((R_BLK, 1), _row)


def _wspec():
    return pl.BlockSpec((D, D), _zero)


def _bspec():
    return pl.BlockSpec((1, D), _zero)


def _tc_y1_body(x_ref, w_ref, d0_ref, d1_ref, o_ref):
    dis = lax.rsqrt(d0_ref[...] + d1_ref[...] + 1.0)
    o_ref[...] = jnp.dot(x_ref[...], w_ref[...],
                         preferred_element_type=jnp.float32) * dis


def _tc_y1(x, W, d0, d1):
    return pl.pallas_call(
        _tc_y1_body,
        grid=(N_P // R_BLK,),
        in_specs=[_rspec(), _wspec(), _cspec(), _cspec()],
        out_specs=_rspec(),
        out_shape=jax.ShapeDtypeStruct((N_P, D), jnp.float32),
    )(x, W, d0, d1)


def _tc_mid_body(s10_ref, s11_ref, y1_ref, d0_ref, d1_ref, a0_ref, a1_ref,
                 c0_ref, c1_ref, x_ref, wl_ref, wr_ref, wg2_ref, bg1_ref,
                 bs1_ref, y2_ref):
    dis = lax.rsqrt(d0_ref[...] + d1_ref[...] + 1.0)
    gcn1 = (s10_ref[...] + s11_ref[...] + y1_ref[...]) * dis + bg1_ref[...]
    mean = (a0_ref[...] + a1_ref[...]) / jnp.maximum(c0_ref[...] + c1_ref[...], 1.0)
    sage = (jnp.dot(mean, wl_ref[...], preferred_element_type=jnp.float32)
            + jnp.dot(x_ref[...], wr_ref[...], preferred_element_type=jnp.float32)
            + bs1_ref[...])
    h = jnp.maximum(gcn1 + sage, 0.0)
    y2_ref[...] = jnp.dot(h, wg2_ref[...],
                          preferred_element_type=jnp.float32) * dis


def _tc_mid(s10, s11, y1, d0, d1, a0, a1, c0, c1, x, wl, wr, wg2, bg1, bs1):
    return pl.pallas_call(
        _tc_mid_body,
        grid=(N_P // R_BLK,),
        in_specs=[_rspec(), _rspec(), _rspec(), _cspec(), _cspec(),
                  _rspec(), _rspec(), _cspec(), _cspec(), _rspec(),
                  _wspec(), _wspec(), _wspec(), _bspec(), _bspec()],
        out_specs=_rspec(),
        out_shape=jax.ShapeDtypeStruct((N_P, D), jnp.float32),
    )(s10, s11, y1, d0, d1, a0, a1, c0, c1, x, wl, wr, wg2, bg1, bs1)


def _tc_out_body(s20_ref, s21_ref, y2_ref, d0_ref, d1_ref, lw_ref, bg2_ref,
                 lb_ref, o_ref):
    dis = lax.rsqrt(d0_ref[...] + d1_ref[...] + 1.0)
    h2 = jnp.maximum((s20_ref[...] + s21_ref[...] + y2_ref[...]) * dis
                     + bg2_ref[...], 0.0)
    o_ref[...] = jnp.dot(h2, lw_ref[...],
                         preferred_element_type=jnp.float32) + lb_ref[...]


def _tc_out(s20, s21, y2, d0, d1, lw, bg2, lb):
    return pl.pallas_call(
        _tc_out_body,
        grid=(N_P // R_BLK,),
        in_specs=[_rspec(), _rspec(), _rspec(), _cspec(), _cspec(),
                  _wspec(), _bspec(), _bspec()],
        out_specs=_rspec(),
        out_shape=jax.ShapeDtypeStruct((N_P, D), jnp.float32),
    )(s20, s21, y2, d0, d1, lw, bg2, lb)


def kernel(x_paper, x_author, edge_index_cites, edge_index_writes,
           W_gcn1, b_gcn1, Wl1, Wr1, b_s1, W_gcn2, b_gcn2, lin_W, lin_b):
    pad_src = jnp.zeros((E_PAD - E,), jnp.int32)
    pad_dst = N_P + jnp.arange(E_PAD - E, dtype=jnp.int32) % (ROWS_PAD - N_P)
    src_c = jnp.concatenate([edge_index_cites[0], pad_src])
    dst_c = jnp.concatenate([edge_index_cites[1], pad_dst])
    src_w = jnp.concatenate([edge_index_writes[0], pad_src])
    dst_w = jnp.concatenate([edge_index_writes[1], pad_dst])

    zrows = jnp.zeros((ROWS_PER_TILE, D), jnp.float32)
    zvec = jnp.zeros((CNT_PER_TILE,), jnp.float32)
    ones = jnp.ones((CHUNK,), jnp.float32)

    deg_p, cnt_p, agg_p = _sc_counts_agg(x_author, dst_c, src_w, dst_w,
                                         zrows, zvec, ones)
    d0 = deg_p[0, :N_P].reshape(N_P, 1)
    d1 = deg_p[1, :N_P].reshape(N_P, 1)
    c0 = cnt_p[0, :N_P].reshape(N_P, 1)
    c1 = cnt_p[1, :N_P].reshape(N_P, 1)

    y1 = _tc_y1(x_paper, W_gcn1, d0, d1)
    s1_p = _sc_scatter_rows(y1, src_c, dst_c, zrows)
    y2 = _tc_mid(s1_p[0, :N_P], s1_p[1, :N_P], y1, d0, d1,
                 agg_p[0, :N_P], agg_p[1, :N_P], c0, c1,
                 x_paper, Wl1, Wr1, W_gcn2,
                 b_gcn1.reshape(1, D), b_s1.reshape(1, D))
    s2_p = _sc_scatter_rows(y2, src_c, dst_c, zrows)
    return _tc_out(s2_p[0, :N_P], s2_p[1, :N_P], y2, d0, d1, lin_W,
                   b_gcn2.reshape(1, D), lin_b.reshape(1, D))


# speedup vs baseline: 1.0742x; 1.0244x over previous
import functools

import jax
import jax.numpy as jnp
from jax import lax
from jax.experimental import pallas as pl
from jax.experimental.pallas import tpu as pltpu
from jax.experimental.pallas import tpu_sc as plsc

N_P = 10000
D = 128
E = 320000

NC = 2
NS = 16
NW = NC * NS
CHUNK = 128
N_CHUNKS = -(-E // CHUNK // NW) * NW
E_PAD = N_CHUNKS * CHUNK
ITERS = N_CHUNKS // NW
N_REAL = -(-E // CHUNK)
ROWS_PAD = 10240
ROWS_PER_TILE = ROWS_PAD // NS
CNT_PAD = 10240
CNT_PER_TILE = CNT_PAD // NS


def _mesh():
    return plsc.VectorSubcoreMesh(core_axis_name="c", subcore_axis_name="s")


def _sc_scatter_rows(y, src, dst, zrows):
    @functools.partial(
        pl.kernel,
        out_type=jax.ShapeDtypeStruct((NC, ROWS_PAD, D), jnp.float32),
        mesh=_mesh(),
        scratch_types=[
            pltpu.VMEM((CHUNK,), jnp.int32),
            pltpu.VMEM((CHUNK,), jnp.int32),
            pltpu.VMEM((CHUNK,), jnp.int32),
            pltpu.VMEM((CHUNK,), jnp.int32),
            pltpu.VMEM((CHUNK, D), jnp.float32),
            pltpu.VMEM((CHUNK, D), jnp.float32),
            pltpu.VMEM_SHARED((ROWS_PAD, D), jnp.float32),
            pltpu.SemaphoreType.DMA,
            pltpu.SemaphoreType.DMA,
            pltpu.SemaphoreType.DMA,
            pltpu.SemaphoreType.DMA,
            pltpu.SemaphoreType.DMA,
            pltpu.SemaphoreType.DMA,
            pltpu.SemaphoreType.DMA,
            pltpu.SemaphoreType.DMA,
        ],
    )
    def run(y_hbm, src_hbm, dst_hbm, z_hbm, out_hbm,
            src_v0, dst_v0, src_v1, dst_v1, rows_v0, rows_v1, acc_sh,
            es0, ed0, es1, ed1, eg0, eg1, ss0, ss1):
        cid = lax.axis_index("c")
        sid = lax.axis_index("s")
        wid = cid * NS + sid
        r0 = sid * ROWS_PER_TILE
        pltpu.sync_copy(z_hbm, acc_sh.at[pl.ds(r0, ROWS_PER_TILE)])
        plsc.subcore_barrier()

        sets = ((src_v0, dst_v0, rows_v0, es0, ed0, eg0, ss0),
                (src_v1, dst_v1, rows_v1, es1, ed1, eg1, ss1))

        def issue_idx(i, b):
            sv, dv, rv, es, ed, eg, ss = sets[b]
            c = i * NW + wid

            @pl.when(c < N_REAL)
            def _():
                base = c * CHUNK
                pltpu.async_copy(src_hbm.at[pl.ds(base, CHUNK)], sv, es)
                pltpu.async_copy(dst_hbm.at[pl.ds(base, CHUNK)], dv, ed)

        def issue_gather(i, b):
            sv, dv, rv, es, ed, eg, ss = sets[b]
            c = i * NW + wid

            @pl.when(c < N_REAL)
            def _():
                base = c * CHUNK
                pltpu.make_async_copy(src_hbm.at[pl.ds(base, CHUNK)], sv, es).wait()
                pltpu.make_async_copy(dst_hbm.at[pl.ds(base, CHUNK)], dv, ed).wait()
                pltpu.async_copy(y_hbm.at[sv], rv, eg)

        def scatter(i, b):
            sv, dv, rv, es, ed, eg, ss = sets[b]
            c = i * NW + wid

            @pl.when(c < N_REAL)
            def _():
                pltpu.make_async_copy(y_hbm.at[sv], rv, eg).wait()
                pltpu.sync_copy(rv, acc_sh.at[dv], add=True)

        issue_idx(0, 0)
        issue_idx(1, 1)
        issue_gather(0, 0)

        @pl.loop(0, ITERS + 1, step=2)
        def _(i):
            issue_gather(i + 1, 1)
            scatter(i, 0)
            issue_idx(i + 2, 0)
            issue_gather(i + 2, 0)
            scatter(i + 1, 1)
            issue_idx(i + 3, 1)

        plsc.subcore_barrier()
        pltpu.sync_copy(acc_sh.at[pl.ds(r0, ROWS_PER_TILE)],
                        out_hbm.at[cid, pl.ds(r0, ROWS_PER_TILE)])

    return run(y, src, dst, zrows)


def _sc_counts_agg(xa, dst_c, src_w, dst_w, zrows, zvec, ones):
    out_types = (
        jax.ShapeDtypeStruct((NC, CNT_PAD), jnp.float32),
        jax.ShapeDtypeStruct((NC, CNT_PAD), jnp.float32),
        jax.ShapeDtypeStruct((NC, ROWS_PAD, D), jnp.float32),
    )

    @functools.partial(
        pl.kernel,
        out_type=out_types,
        mesh=_mesh(),
        scratch_types=[
            pltpu.VMEM((CHUNK,), jnp.int32),
            pltpu.VMEM((CHUNK,), jnp.int32),
            pltpu.VMEM((CHUNK,), jnp.int32),
            pltpu.VMEM((CHUNK,), jnp.int32),
            pltpu.VMEM((CHUNK,), jnp.int32),
            pltpu.VMEM((CHUNK,), jnp.int32),
            pltpu.VMEM((CHUNK, D), jnp.float32),
            pltpu.VMEM((CHUNK, D), jnp.float32),
            pltpu.VMEM((CHUNK,), jnp.float32),
            pltpu.VMEM_SHARED((CNT_PAD,), jnp.float32),
            pltpu.VMEM_SHARED((CNT_PAD,), jnp.float32),
            pltpu.VMEM_SHARED((ROWS_PAD, D), jnp.float32),
            pltpu.SemaphoreType.DMA,
            pltpu.SemaphoreType.DMA,
            pltpu.SemaphoreType.DMA,
            pltpu.SemaphoreType.DMA,
            pltpu.SemaphoreType.DMA,
            pltpu.SemaphoreType.DMA,
            pltpu.SemaphoreType.DMA,
            pltpu.SemaphoreType.DMA,
        ],
    )
    def run(xa_hbm, dstc_hbm, srcw_hbm, dstw_hbm, z_hbm, zv_hbm, ones_hbm,
            odeg_hbm, ocnt_hbm, oagg_hbm,
            dca0, swv0, dwv0, dca1, swv1, dwv1, rows_v0, rows_v1, ones_v,
            acc_deg, acc_cnt, acc_agg,
            ea0, eb0, ec0, ea1, eb1, ec1, eg0, eg1):
        cid = lax.axis_index("c")
        sid = lax.axis_index("s")
        wid = cid * NS + sid
        r0 = sid * ROWS_PER_TILE
        c0 = sid * CNT_PER_TILE
        pltpu.sync_copy(ones_hbm, ones_v)
        pltpu.sync_copy(z_hbm, acc_agg.at[pl.ds(r0, ROWS_PER_TILE)])
        pltpu.sync_copy(zv_hbm, acc_deg.at[pl.ds(c0, CNT_PER_TILE)])
        pltpu.sync_copy(zv_hbm, acc_cnt.at[pl.ds(c0, CNT_PER_TILE)])
        plsc.subcore_barrier()

        sets = ((dca0, swv0, dwv0, rows_v0, ea0, eb0, ec0, eg0),
                (dca1, swv1, dwv1, rows_v1, ea1, eb1, ec1, eg1))

        def issue_idx(i, b):
            dca, swv, dwv, rv, ea, eb, ec, eg = sets[b]
            c = i * NW + wid

            @pl.when(c < N_REAL)
            def _():
                base = c * CHUNK
                pltpu.async_copy(dstc_hbm.at[pl.ds(base, CHUNK)], dca, ea)
                pltpu.async_copy(srcw_hbm.at[pl.ds(base, CHUNK)], swv, eb)
                pltpu.async_copy(dstw_hbm.at[pl.ds(base, CHUNK)], dwv, ec)

        def issue_gather(i, b):
            dca, swv, dwv, rv, ea, eb, ec, eg = sets[b]
            c = i * NW + wid

            @pl.when(c < N_REAL)
            def _():
                base = c * CHUNK
                pltpu.make_async_copy(srcw_hbm.at[pl.ds(base, CHUNK)], swv, eb).wait()
                pltpu.async_copy(xa_hbm.at[swv], rv, eg)

        def deg_scatter(i, b):
            dca, swv, dwv, rv, ea, eb, ec, eg = sets[b]
            c = i * NW + wid

            @pl.when(c < N_REAL)
            def _():
                base = c * CHUNK
                pltpu.make_async_copy(dstc_hbm.at[pl.ds(base, CHUNK)], dca, ea).wait()
                pltpu.sync_copy(ones_v, acc_deg.at[dca], add=True)

        def work(i, b):
            dca, swv, dwv, rv, ea, eb, ec, eg = sets[b]
            c = i * NW + wid

            @pl.when(c < N_REAL)
            def _():
                base = c * CHUNK
                pltpu.make_async_copy(dstw_hbm.at[pl.ds(base, CHUNK)], dwv, ec).wait()
                pltpu.make_async_copy(xa_hbm.at[swv], rv, eg).wait()
                pltpu.sync_copy(rv, acc_agg.at[dwv], add=True)
                pltpu.sync_copy(ones_v, acc_cnt.at[dwv], add=True)

        issue_idx(0, 0)
        issue_idx(1, 1)
        issue_gather(0, 0)

        @pl.loop(0, ITERS + 1, step=2)
        def _(i):
            issue_gather(i + 1, 1)
            deg_scatter(i, 0)
            work(i, 0)
            issue_idx(i + 2, 0)
            issue_gather(i + 2, 0)
            deg_scatter(i + 1, 1)
            work(i + 1, 1)
            issue_idx(i + 3, 1)

        plsc.subcore_barrier()
        pltpu.sync_copy(acc_deg.at[pl.ds(c0, CNT_PER_TILE)],
                        odeg_hbm.at[cid, pl.ds(c0, CNT_PER_TILE)])
        pltpu.sync_copy(acc_cnt.at[pl.ds(c0, CNT_PER_TILE)],
                        ocnt_hbm.at[cid, pl.ds(c0, CNT_PER_TILE)])
        pltpu.sync_copy(acc_agg.at[pl.ds(r0, ROWS_PER_TILE)],
                        oagg_hbm.at[cid, pl.ds(r0, ROWS_PER_TILE)])

    return run(xa, dst_c, src_w, dst_w, zrows, zvec, ones)


R_BLK = 2000


def _row(i):
    return (i, 0)


def _zero(i):
    return (0, 0)


def _rspec():
    return pl.BlockSpec((R_BLK, D), _row)


def _cspec():
    return pl.BlockSpec((R_BLK, 1), _row)


def _wspec():
    return pl.BlockSpec((D, D), _zero)


def _bspec():
    return pl.BlockSpec((1, D), _zero)


def _tc_y1_body(x_ref, w_ref, d0_ref, d1_ref, o_ref):
    dis = lax.rsqrt(d0_ref[...] + d1_ref[...] + 1.0)
    o_ref[...] = jnp.dot(x_ref[...], w_ref[...],
                         preferred_element_type=jnp.float32) * dis


def _tc_y1(x, W, d0, d1):
    return pl.pallas_call(
        _tc_y1_body,
        grid=(N_P // R_BLK,),
        in_specs=[_rspec(), _wspec(), _cspec(), _cspec()],
        out_specs=_rspec(),
        out_shape=jax.ShapeDtypeStruct((N_P, D), jnp.float32),
    )(x, W, d0, d1)


def _pspec(core):
    return pl.BlockSpec((1, R_BLK, D), lambda i, c=core: (c, i, 0))


def _tc_mid_body(s10_ref, s11_ref, y1_ref, d0_ref, d1_ref, a0_ref, a1_ref,
                 c0_ref, c1_ref, x_ref, wl_ref, wr_ref, wg2_ref, bg1_ref,
                 bs1_ref, y2_ref):
    dis = lax.rsqrt(d0_ref[...] + d1_ref[...] + 1.0)
    gcn1 = (s10_ref[0] + s11_ref[0] + y1_ref[...]) * dis + bg1_ref[...]
    mean = (a0_ref[0] + a1_ref[0]) / jnp.maximum(c0_ref[...] + c1_ref[...], 1.0)
    sage = (jnp.dot(mean, wl_ref[...], preferred_element_type=jnp.float32)
            + jnp.dot(x_ref[...], wr_ref[...], preferred_element_type=jnp.float32)
            + bs1_ref[...])
    h = jnp.maximum(gcn1 + sage, 0.0)
    y2_ref[...] = jnp.dot(h, wg2_ref[...],
                          preferred_element_type=jnp.float32) * dis


def _tc_mid(s1_p, y1, d0, d1, agg_p, c0, c1, x, wl, wr, wg2, bg1, bs1):
    return pl.pallas_call(
        _tc_mid_body,
        grid=(N_P // R_BLK,),
        in_specs=[_pspec(0), _pspec(1), _rspec(), _cspec(), _cspec(),
                  _pspec(0), _pspec(1), _cspec(), _cspec(), _rspec(),
                  _wspec(), _wspec(), _wspec(), _bspec(), _bspec()],
        out_specs=_rspec(),
        out_shape=jax.ShapeDtypeStruct((N_P, D), jnp.float32),
    )(s1_p, s1_p, y1, d0, d1, agg_p, agg_p, c0, c1, x, wl, wr, wg2, bg1, bs1)


def _tc_out_body(s20_ref, s21_ref, y2_ref, d0_ref, d1_ref, lw_ref, bg2_ref,
                 lb_ref, o_ref):
    dis = lax.rsqrt(d0_ref[...] + d1_ref[...] + 1.0)
    h2 = jnp.maximum((s20_ref[0] + s21_ref[0] + y2_ref[...]) * dis
                     + bg2_ref[...], 0.0)
    o_ref[...] = jnp.dot(h2, lw_ref[...],
                         preferred_element_type=jnp.float32) + lb_ref[...]


def _tc_out(s2_p, y2, d0, d1, lw, bg2, lb):
    return pl.pallas_call(
        _tc_out_body,
        grid=(N_P // R_BLK,),
        in_specs=[_pspec(0), _pspec(1), _rspec(), _cspec(), _cspec(),
                  _wspec(), _bspec(), _bspec()],
        out_specs=_rspec(),
        out_shape=jax.ShapeDtypeStruct((N_P, D), jnp.float32),
    )(s2_p, s2_p, y2, d0, d1, lw, bg2, lb)


def kernel(x_paper, x_author, edge_index_cites, edge_index_writes,
           W_gcn1, b_gcn1, Wl1, Wr1, b_s1, W_gcn2, b_gcn2, lin_W, lin_b):
    src_c = edge_index_cites[0]
    dst_c = edge_index_cites[1]
    src_w = edge_index_writes[0]
    dst_w = edge_index_writes[1]

    zrows = jnp.zeros((ROWS_PER_TILE, D), jnp.float32)
    zvec = jnp.zeros((CNT_PER_TILE,), jnp.float32)
    ones = jnp.ones((CHUNK,), jnp.float32)

    deg_p, cnt_p, agg_p = _sc_counts_agg(x_author, dst_c, src_w, dst_w,
                                         zrows, zvec, ones)
    d0 = deg_p[0, :N_P].reshape(N_P, 1)
    d1 = deg_p[1, :N_P].reshape(N_P, 1)
    c0 = cnt_p[0, :N_P].reshape(N_P, 1)
    c1 = cnt_p[1, :N_P].reshape(N_P, 1)

    y1 = _tc_y1(x_paper, W_gcn1, d0, d1)
    s1_p = _sc_scatter_rows(y1, src_c, dst_c, zrows)
    y2 = _tc_mid(s1_p, y1, d0, d1, agg_p, c0, c1,
                 x_paper, Wl1, Wr1, W_gcn2,
                 b_gcn1.reshape(1, D), b_s1.reshape(1, D))
    s2_p = _sc_scatter_rows(y2, src_c, dst_c, zrows)
    return _tc_out(s2_p, y2, d0, d1, lin_W,
                   b_gcn2.reshape(1, D), lin_b.reshape(1, D))
